# Initial kernel scaffold; baseline (speedup 1.0000x reference)
import functools

import jax
import jax.numpy as jnp
from jax import lax
from jax.experimental import pallas as pl
from jax.experimental.pallas import tpu as pltpu
from jax.experimental.pallas import tpu_sc as plsc

NC = 2
NS = 16
L = 16
G = 16
K = 80
BS = 1024

THETA = (0.5, 0.3, 0.1, 0.05, 0.05)


def _sc_degree(npad, ept, k):
    nchunks = ept // k
    rpt = npad // NS
    mesh = plsc.VectorSubcoreMesh(core_axis_name="c", subcore_axis_name="s")

    @functools.partial(
        pl.kernel,
        mesh=mesh,
        out_type=jax.ShapeDtypeStruct((2 * npad, G), jnp.float32),
        scratch_types=[
            pltpu.VMEM((1, k), jnp.int32),
            pltpu.VMEM((k, G), jnp.float32),
            pltpu.VMEM((k, G), jnp.float32),
            pltpu.VMEM_SHARED((npad, G), jnp.float32),
            pltpu.SemaphoreType.DMA,
        ],
    )
    def deg_kernel(row_hbm, out_hbm, idx_v, ones_v, z_v, acc_sh, sem):
        cid = lax.axis_index("c")
        sid = lax.axis_index("s")

        @pl.loop(0, k)
        def _(i):
            ones_v[i, :] = jnp.full((G,), 1.0, jnp.float32)
            z_v[i, :] = jnp.zeros((G,), jnp.float32)

        r0 = sid * rpt

        @pl.loop(0, rpt, step=k)
        def _(r):
            pltpu.sync_copy(z_v, acc_sh.at[pl.ds(r0 + r, k)])

        plsc.subcore_barrier()

        base = (cid * NS + sid) * ept

        @pl.loop(0, nchunks)
        def _(i):
            pltpu.sync_copy(row_hbm.at[pl.ds(base + i * k, k)], idx_v.at[0])
            pltpu.sync_copy(ones_v, acc_sh.at[idx_v.at[0]], add=True)

        plsc.subcore_barrier()
        pltpu.sync_copy(acc_sh.at[pl.ds(r0, rpt)],
                        out_hbm.at[pl.ds(cid * npad + r0, rpt)])

    return deg_kernel


def _sc_apply(npad, d, ept, k):
    nchunks = ept // k
    rpt = npad // NS
    mesh = plsc.VectorSubcoreMesh(core_axis_name="c", subcore_axis_name="s")

    @functools.partial(
        pl.kernel,
        mesh=mesh,
        out_type=jax.ShapeDtypeStruct((2 * npad, d), jnp.float32),
        scratch_types=[
            pltpu.VMEM((1, k), jnp.int32),
            pltpu.VMEM((1, k), jnp.int32),
            pltpu.VMEM((k, d), jnp.float32),
            pltpu.VMEM((k, d), jnp.float32),
            pltpu.VMEM_SHARED((npad, d), jnp.float32),
            pltpu.SemaphoreType.DMA,
        ],
    )
    def apply_kernel(s_hbm, row_hbm, col_hbm, out_hbm,
                     cidx_v, ridx_v, rows_v, z_v, acc_sh, sem):
        cid = lax.axis_index("c")
        sid = lax.axis_index("s")

        @pl.loop(0, k)
        def _(i):
            @pl.loop(0, d, step=L)
            def _(j):
                z_v[i, pl.ds(j, L)] = jnp.zeros((L,), jnp.float32)

        r0 = sid * rpt

        @pl.loop(0, rpt, step=k)
        def _(r):
            pltpu.sync_copy(z_v, acc_sh.at[pl.ds(r0 + r, k)])

        plsc.subcore_barrier()

        base = (cid * NS + sid) * ept

        @pl.loop(0, nchunks)
        def _(i):
            pltpu.sync_copy(col_hbm.at[pl.ds(base + i * k, k)], cidx_v.at[0])
            pltpu.sync_copy(row_hbm.at[pl.ds(base + i * k, k)], ridx_v.at[0])
            pltpu.async_copy(s_hbm.at[cidx_v.at[0]], rows_v, sem).wait()
            pltpu.sync_copy(rows_v, acc_sh.at[ridx_v.at[0]], add=True)

        plsc.subcore_barrier()
        pltpu.sync_copy(acc_sh.at[pl.ds(r0, rpt)],
                        out_hbm.at[pl.ds(cid * npad + r0, rpt)])

    return apply_kernel


def _tc_init(npad, d, theta0):
    nb = npad // BS

    def body(h_ref, d0_ref, d1_ref, dinv_ref, s_ref, oa_ref):
        deg = d0_ref[...] + d1_ref[...]
        dinv = jnp.where(deg > 0, lax.rsqrt(deg), 0.0)
        dinv_ref[...] = dinv
        hb = h_ref[...]
        s_ref[...] = dinv[:, :1] * hb
        oa_ref[...] = theta0 * hb

    return pl.pallas_call(
        body,
        grid=(nb,),
        in_specs=[
            pl.BlockSpec((BS, d), lambda i: (i, 0)),
            pl.BlockSpec((BS, G), lambda i: (i, 0)),
            pl.BlockSpec((BS, G), lambda i: (i + nb, 0)),
        ],
        out_specs=[
            pl.BlockSpec((BS, G), lambda i: (i, 0)),
            pl.BlockSpec((BS, d), lambda i: (i, 0)),
            pl.BlockSpec((BS, d), lambda i: (i, 0)),
        ],
        out_shape=[
            jax.ShapeDtypeStruct((npad, G), jnp.float32),
            jax.ShapeDtypeStruct((npad, d), jnp.float32),
            jax.ShapeDtypeStruct((npad, d), jnp.float32),
        ],
    )


def _tc_combine(npad, d, theta_k):
    nb = npad // BS

    def body(f_ref, p0_ref, p1_ref, dinv_ref, oa_ref,
             fn_ref, oan_ref, sn_ref):
        dinv = dinv_ref[...][:, :1]
        fn = f_ref[...] - dinv * (p0_ref[...] + p1_ref[...])
        fn_ref[...] = fn
        oan_ref[...] = oa_ref[...] + theta_k * fn
        sn_ref[...] = dinv * fn

    return pl.pallas_call(
        body,
        grid=(nb,),
        in_specs=[
            pl.BlockSpec((BS, d), lambda i: (i, 0)),
            pl.BlockSpec((BS, d), lambda i: (i, 0)),
            pl.BlockSpec((BS, d), lambda i: (i + nb, 0)),
            pl.BlockSpec((BS, G), lambda i: (i, 0)),
            pl.BlockSpec((BS, d), lambda i: (i, 0)),
        ],
        out_specs=[
            pl.BlockSpec((BS, d), lambda i: (i, 0)),
            pl.BlockSpec((BS, d), lambda i: (i, 0)),
            pl.BlockSpec((BS, d), lambda i: (i, 0)),
        ],
        out_shape=[
            jax.ShapeDtypeStruct((npad, d), jnp.float32),
            jax.ShapeDtypeStruct((npad, d), jnp.float32),
            jax.ShapeDtypeStruct((npad, d), jnp.float32),
        ],
    )


def kernel(h, edge_index):
    n, d = h.shape
    e = edge_index.shape[1]

    npad = ((n + NS * K - 1) // (NS * K)) * (NS * K)
    echunk = NC * NS * K
    epad = ((e + echunk - 1) // echunk) * echunk
    ept = epad // (NC * NS)

    row = edge_index[0]
    col = edge_index[1]
    if epad != e:
        row = jnp.concatenate(
            [row, jnp.full((epad - e,), npad - 1, jnp.int32)])
        col = jnp.concatenate([col, jnp.zeros((epad - e,), jnp.int32)])
    h_pad = jnp.pad(h, ((0, npad - n), (0, 0))) if npad != n else h

    deg_p = _sc_degree(npad, ept, K)(row)
    dinv, s, out = _tc_init(npad, d, THETA[0])(h_pad, deg_p, deg_p)

    sc_apply = _sc_apply(npad, d, ept, K)
    feat = h_pad
    for kk in range(1, len(THETA)):
        part = sc_apply(s, row, col)
        feat, out, s = _tc_combine(npad, d, THETA[kk])(
            feat, part, part, dinv, out)

    return out[:n]


# speedup vs baseline: 7.3771x; 7.3771x over previous
import dataclasses
import functools

import jax
import jax.numpy as jnp
from jax import lax
from jax.experimental import pallas as pl
from jax.experimental.pallas import tpu as pltpu
from jax.experimental.pallas import tpu_sc as plsc

NC = 2
NS = 16
L = 16
K = 80
BS = 1024

THETA = (0.5, 0.3, 0.1, 0.05, 0.05)

_SC_PARAMS = pltpu.CompilerParams()
if "needs_layout_passes" in pltpu.CompilerParams.__dataclass_fields__:
    _SC_PARAMS = dataclasses.replace(_SC_PARAMS, needs_layout_passes=False)


def _sc_degree(npad, ept, k):
    nr = npad // 128
    nchunks = ept // k
    mesh = plsc.VectorSubcoreMesh(core_axis_name="c", subcore_axis_name="s")

    @functools.partial(
        pl.kernel,
        mesh=mesh,
        compiler_params=_SC_PARAMS,
        out_type=jax.ShapeDtypeStruct((2 * nr, 128), jnp.float32),
        scratch_types=[
            pltpu.VMEM((1, k), jnp.int32),
            pltpu.VMEM((nr, 128), jnp.float32),
            pltpu.VMEM((1, nr), jnp.int32),
            pltpu.VMEM_SHARED((nr, 128), jnp.float32),
            pltpu.SemaphoreType.DMA,
        ],
    )
    def deg_kernel(row_hbm, out_hbm, idx_v, hist_v, iden_v, acc_sh, sem):
        cid = lax.axis_index("c")
        sid = lax.axis_index("s")

        @pl.loop(0, nr)
        def _(i):
            @pl.loop(0, 128, step=L)
            def _(j):
                hist_v[i, pl.ds(j, L)] = jnp.zeros((L,), jnp.float32)

        @pl.loop(0, nr, step=L)
        def _(i):
            iden_v[0, pl.ds(i, L)] = lax.iota(jnp.int32, L) + i

        @pl.when(sid < nr // 8)
        def _():
            pltpu.sync_copy(hist_v.at[pl.ds(sid * 8, 8)],
                            acc_sh.at[pl.ds(sid * 8, 8)])
        plsc.subcore_barrier()

        base = (cid * NS + sid) * ept
        ones16 = jnp.full((L,), 1.0, jnp.float32)

        @pl.loop(0, nchunks)
        def _(i):
            pltpu.sync_copy(row_hbm.at[pl.ds(base + i * k, k)], idx_v.at[0])

            @pl.loop(0, k, step=L)
            def _(j):
                idx = idx_v[0, pl.ds(j, L)]
                r = lax.shift_right_logical(idx, 7)
                c = lax.bitwise_and(idx, 127)
                plsc.addupdate_scatter(hist_v, [r, c], ones16)

        pltpu.sync_copy(hist_v, acc_sh.at[iden_v.at[0]], add=True)
        plsc.subcore_barrier()

        @pl.when(sid < nr // 8)
        def _():
            pltpu.sync_copy(acc_sh.at[pl.ds(sid * 8, 8)],
                            out_hbm.at[pl.ds(cid * nr + sid * 8, 8)])

    return deg_kernel


def _sc_apply(npad, d, ept, k):
    nchunks = ept // k
    rpt = npad // NS
    mesh = plsc.VectorSubcoreMesh(core_axis_name="c", subcore_axis_name="s")

    @functools.partial(
        pl.kernel,
        mesh=mesh,
        compiler_params=_SC_PARAMS,
        out_type=jax.ShapeDtypeStruct((2 * npad, d), jnp.float32),
        scratch_types=[
            pltpu.VMEM((1, k), jnp.int32),
            pltpu.VMEM((1, k), jnp.int32),
            pltpu.VMEM((k, d), jnp.float32),
            pltpu.VMEM((k, d), jnp.float32),
            pltpu.VMEM_SHARED((npad, d), jnp.float32),
            pltpu.SemaphoreType.DMA,
        ],
    )
    def apply_kernel(s_hbm, row_hbm, col_hbm, out_hbm,
                     cidx_v, ridx_v, rows_v, z_v, acc_sh, sem):
        cid = lax.axis_index("c")
        sid = lax.axis_index("s")

        @pl.loop(0, k)
        def _(i):
            @pl.loop(0, d, step=L)
            def _(j):
                z_v[i, pl.ds(j, L)] = jnp.zeros((L,), jnp.float32)

        r0 = sid * rpt

        @pl.loop(0, rpt, step=k)
        def _(r):
            pltpu.sync_copy(z_v, acc_sh.at[pl.ds(r0 + r, k)])

        plsc.subcore_barrier()

        base = (cid * NS + sid) * ept

        @pl.loop(0, nchunks)
        def _(i):
            pltpu.sync_copy(col_hbm.at[pl.ds(base + i * k, k)], cidx_v.at[0])
            pltpu.sync_copy(row_hbm.at[pl.ds(base + i * k, k)], ridx_v.at[0])
            pltpu.async_copy(s_hbm.at[cidx_v.at[0]], rows_v, sem).wait()
            pltpu.sync_copy(rows_v, acc_sh.at[ridx_v.at[0]], add=True)

        plsc.subcore_barrier()
        pltpu.sync_copy(acc_sh.at[pl.ds(r0, rpt)],
                        out_hbm.at[pl.ds(cid * npad + r0, rpt)])

    return apply_kernel


def _tc_init(npad, d, theta0):
    nb = npad // BS
    nr = npad // 128
    rpb = BS // 128

    def body(h_ref, d0_ref, d1_ref, dinv_ref, s_ref, oa_ref):
        deg = d0_ref[...] + d1_ref[...]
        dinv8 = jnp.where(deg > 0, lax.rsqrt(deg), 0.0)
        jrow = lax.broadcasted_iota(jnp.int32, (BS, rpb), 0) // 128
        sel = (jrow == lax.broadcasted_iota(jnp.int32, (BS, rpb), 1))
        spread = jax.lax.dot_general(
            sel.astype(jnp.float32), dinv8,
            dimension_numbers=(((1,), (0,)), ((), ())),
            preferred_element_type=jnp.float32)
        jcol = lax.broadcasted_iota(jnp.int32, (BS, 128), 0) % 128
        mask = (jcol == lax.broadcasted_iota(jnp.int32, (BS, 128), 1))
        dinv_col = jnp.sum(jnp.where(mask, spread, 0.0), axis=1,
                           keepdims=True)
        dinv_blk = lax.broadcast_in_dim(dinv_col, (BS, d), (0, 1))
        dinv_ref[...] = dinv_blk
        hb = h_ref[...]
        s_ref[...] = dinv_blk * hb
        oa_ref[...] = theta0 * hb

    return pl.pallas_call(
        body,
        grid=(nb,),
        in_specs=[
            pl.BlockSpec((BS, d), lambda i: (i, 0)),
            pl.BlockSpec((rpb, 128), lambda i: (i, 0)),
            pl.BlockSpec((rpb, 128), lambda i: (i + nb, 0)),
        ],
        out_specs=[
            pl.BlockSpec((BS, d), lambda i: (i, 0)),
            pl.BlockSpec((BS, d), lambda i: (i, 0)),
            pl.BlockSpec((BS, d), lambda i: (i, 0)),
        ],
        out_shape=[
            jax.ShapeDtypeStruct((npad, d), jnp.float32),
            jax.ShapeDtypeStruct((npad, d), jnp.float32),
            jax.ShapeDtypeStruct((npad, d), jnp.float32),
        ],
    )


def _tc_combine(npad, d, theta_k):
    nb = npad // BS

    def body(f_ref, p0_ref, p1_ref, dinv_ref, oa_ref,
             fn_ref, oan_ref, sn_ref):
        dinv = dinv_ref[...]
        fn = f_ref[...] - dinv * (p0_ref[...] + p1_ref[...])
        fn_ref[...] = fn
        oan_ref[...] = oa_ref[...] + theta_k * fn
        sn_ref[...] = dinv * fn

    return pl.pallas_call(
        body,
        grid=(nb,),
        in_specs=[
            pl.BlockSpec((BS, d), lambda i: (i, 0)),
            pl.BlockSpec((BS, d), lambda i: (i, 0)),
            pl.BlockSpec((BS, d), lambda i: (i + nb, 0)),
            pl.BlockSpec((BS, d), lambda i: (i, 0)),
            pl.BlockSpec((BS, d), lambda i: (i, 0)),
        ],
        out_specs=[
            pl.BlockSpec((BS, d), lambda i: (i, 0)),
            pl.BlockSpec((BS, d), lambda i: (i, 0)),
            pl.BlockSpec((BS, d), lambda i: (i, 0)),
        ],
        out_shape=[
            jax.ShapeDtypeStruct((npad, d), jnp.float32),
            jax.ShapeDtypeStruct((npad, d), jnp.float32),
            jax.ShapeDtypeStruct((npad, d), jnp.float32),
        ],
    )


def kernel(h, edge_index):
    n, d = h.shape
    e = edge_index.shape[1]

    npad = ((n + NS * K - 1) // (NS * K)) * (NS * K)
    echunk = NC * NS * K
    epad = ((e + echunk - 1) // echunk) * echunk
    ept = epad // (NC * NS)

    row = edge_index[0]
    col = edge_index[1]
    if epad != e:
        row = jnp.concatenate(
            [row, jnp.full((epad - e,), npad - 1, jnp.int32)])
        col = jnp.concatenate([col, jnp.zeros((epad - e,), jnp.int32)])
    h_pad = jnp.pad(h, ((0, npad - n), (0, 0))) if npad != n else h

    deg_p = _sc_degree(npad, ept, K)(row)
    dinv, s, out = _tc_init(npad, d, THETA[0])(h_pad, deg_p, deg_p)

    sc_apply = _sc_apply(npad, d, ept, K)
    feat = h_pad
    for kk in range(1, len(THETA)):
        part = sc_apply(s, row, col)
        feat, out, s = _tc_combine(npad, d, THETA[kk])(
            feat, part, part, dinv, out)

    return out[:n]
